# baseline (speedup 1.0000x reference)
import functools

import jax
import jax.numpy as jnp
import numpy as np
from jax import lax
from jax.experimental import pallas as pl
from jax.experimental.pallas import tpu as pltpu
from jax.experimental.pallas import tpu_sc as plsc

N = 4096
K = 128
LANES = 16
NC, NS = 2, 16
NW = NC * NS
ROWS_PER_W = N // NW
G = 8
NGROUPS = ROWS_PER_W // G
STRIDE = N + 2 * LANES
STEPS = N // 2
UNROLL = 8
HBINS = 257

_SIGN = np.uint32(0x80000000)


def _sc_body(s_hbm, tot_hbm, diag_hbm, stag, buf, ebuf, hist_c, hist_f,
             tot_res, diag_res, sem):
    wid = lax.axis_index("s") * NC + lax.axis_index("c")
    row0 = wid * ROWS_PER_W
    lane_iota = lax.iota(jnp.int32, LANES)
    i8 = lane_iota & 7
    half = lane_iota >> 3
    ones_i = jnp.ones((LANES,), jnp.int32)
    zeros_i = jnp.zeros((LANES,), jnp.int32)
    zeros_f = jnp.zeros((LANES,), jnp.float32)
    hscat = lane_iota * HBINS
    hbase = i8 * HBINS
    hbase2 = (i8 + 8) * HBINS
    idx0 = i8 * (STRIDE + 2) + half

    def do_group(g, _):
        base_row = row0 + g * G

        pltpu.sync_copy(s_hbm.at[pl.ds(base_row, G)], stag)

        dval = plsc.load_gather(stag, [i8, i8 + base_row])
        ediag = jnp.exp(dval)

        for l in range(G):
            grow = base_row + l

            def prep_step(j, carry, l=l, grow=grow):
                for k in range(UNROLL):
                    c0 = (j * UNROLL + k) * LANES
                    o = l * STRIDE + c0
                    col = lane_iota + c0
                    v = stag[l, pl.ds(c0, LANES)]
                    isdiag = col == grow
                    b = lax.bitcast_convert_type(v, jnp.uint32)
                    u = jnp.where(b >= _SIGN, ~b, b | _SIGN)
                    u = jnp.where(isdiag, jnp.uint32(0), u)
                    buf[pl.ds(o, LANES)] = lax.bitcast_convert_type(
                        u, jnp.float32)
                    ebuf[pl.ds(o, LANES)] = jnp.exp(v)
                return carry

            lax.fori_loop(0, N // (LANES * UNROLL), prep_step,
                          0, unroll=False)

        for l in range(1, G):
            fu = buf[pl.ds(l * STRIDE, LANES)]
            fe = ebuf[pl.ds(l * STRIDE, LANES)]
            tm = lane_iota < 2 * l
            plsc.store_scatter(buf, [lane_iota + (l * STRIDE + N)], fu,
                               mask=tm)
            plsc.store_scatter(ebuf, [lane_iota + (l * STRIDE + N)], fe,
                               mask=tm)

        prefix = jnp.zeros((LANES,), jnp.uint32)
        k_rem = jnp.full((LANES,), K, jnp.int32)
        s_total = zeros_f

        for lvl in range(4):
            shift = np.uint32(24 - 8 * lvl)
            upper = np.uint32(0xFFFFFFFF) ^ np.uint32(
                (1 << (32 - 8 * lvl)) - 1)

            def zh(i, c):
                hist_c[pl.ds(i * LANES, LANES)] = zeros_i
                hist_f[pl.ds(i * LANES, LANES)] = zeros_f
                return c

            lax.fori_loop(0, (LANES * HBINS) // LANES, zh, 0, unroll=False)

            def hist_step(j, idx, prefix=prefix, upper=upper, shift=shift,
                          lvl=lvl):
                for k in range(UNROLL):
                    u = lax.bitcast_convert_type(
                        plsc.load_gather(buf, [idx]), jnp.uint32)
                    ev = plsc.load_gather(ebuf, [idx])
                    byte = lax.shift_right_logical(u, shift) & jnp.uint32(0xFF)
                    hidx = hscat + byte.astype(jnp.int32)
                    if lvl == 0:
                        plsc.addupdate_scatter(hist_c, [hidx], ones_i)
                        plsc.addupdate_scatter(hist_f, [hidx], ev)
                    else:
                        act = (u & upper) == prefix
                        plsc.addupdate_scatter(hist_c, [hidx], ones_i,
                                               mask=act)
                        plsc.addupdate_scatter(hist_f, [hidx], ev, mask=act)
                    idx = idx + 2
                return idx

            lax.fori_loop(0, STEPS // UNROLL, hist_step, idx0, unroll=False)

            def scan_step(i, carry):
                acc_c, acc_e, B, gt, s_lvl, done = carry
                for k in range(4):
                    b = 255 - (i * 4 + k)
                    h = (plsc.load_gather(hist_c, [hbase + b])
                         + plsc.load_gather(hist_c, [hbase2 + b]))
                    he = (plsc.load_gather(hist_f, [hbase + b])
                          + plsc.load_gather(hist_f, [hbase2 + b]))
                    acc_c = acc_c + h
                    crossed = acc_c >= k_rem
                    newly = jnp.logical_and(crossed, jnp.logical_not(done))
                    B = jnp.where(newly, b, B)
                    gt = jnp.where(newly, acc_c - h, gt)
                    s_lvl = jnp.where(newly, acc_e, s_lvl)
                    acc_e = acc_e + he
                    done = jnp.logical_or(done, crossed)
                return acc_c, acc_e, B, gt, s_lvl, done

            _, _, B, gt, s_lvl, _ = lax.fori_loop(
                0, 256 // 4, scan_step,
                (zeros_i, zeros_f, zeros_i, zeros_i, zeros_f,
                 jnp.zeros((LANES,), jnp.bool_)), unroll=False)

            prefix = prefix | lax.shift_left(
                B.astype(jnp.uint32), shift)
            k_rem = k_rem - gt
            s_total = s_total + s_lvl

        t_bits = jnp.where(prefix >= _SIGN, prefix & ~_SIGN, ~prefix)
        t_val = lax.bitcast_convert_type(t_bits, jnp.float32)
        neg = s_total + k_rem.astype(jnp.float32) * jnp.exp(t_val)
        total = neg + ediag

        out_idx = i8 + g * G
        m8 = lane_iota < 8
        plsc.store_scatter(tot_res, [out_idx], total, mask=m8)
        plsc.store_scatter(diag_res, [out_idx], dval, mask=m8)
        return 0

    lax.fori_loop(0, NGROUPS, do_group, 0, unroll=False)
    pltpu.sync_copy(tot_res, tot_hbm.at[pl.ds(row0, ROWS_PER_W)])
    pltpu.sync_copy(diag_res, diag_hbm.at[pl.ds(row0, ROWS_PER_W)])


@jax.jit
def _sc_select(s):
    mesh = plsc.VectorSubcoreMesh(core_axis_name="c", subcore_axis_name="s",
                                  num_cores=NC, num_subcores=NS)
    return pl.kernel(
        _sc_body,
        out_type=[
            jax.ShapeDtypeStruct((N,), jnp.float32),
            jax.ShapeDtypeStruct((N,), jnp.float32),
        ],
        mesh=mesh,
        compiler_params=pltpu.CompilerParams(needs_layout_passes=False),
        scratch_types=[
            pltpu.VMEM((G, N), jnp.float32),
            pltpu.VMEM((G * STRIDE,), jnp.float32),
            pltpu.VMEM((G * STRIDE,), jnp.float32),
            pltpu.VMEM((LANES * HBINS,), jnp.int32),
            pltpu.VMEM((LANES * HBINS,), jnp.float32),
            pltpu.VMEM((ROWS_PER_W,), jnp.float32),
            pltpu.VMEM((ROWS_PER_W,), jnp.float32),
            pltpu.SemaphoreType.DMA,
        ],
    )(s)


def _finish_body(tot_ref, diag_ref, out_ref):
    out_ref[0, 0] = jnp.mean(jnp.log(tot_ref[...]) - diag_ref[...])


@jax.jit
def _tc_finish(tot, diag):
    return pl.pallas_call(
        _finish_body,
        out_shape=jax.ShapeDtypeStruct((1, 1), jnp.float32),
        out_specs=pl.BlockSpec(memory_space=pltpu.SMEM),
    )(tot, diag)


def kernel(s):
    tot, diag = _sc_select(s)
    out = _tc_finish(tot.reshape(32, ROWS_PER_W), diag.reshape(32, ROWS_PER_W))
    return out[0, 0]


# speedup vs baseline: 3.1359x; 3.1359x over previous
import functools

import jax
import jax.numpy as jnp
import numpy as np
from jax import lax
from jax.experimental import pallas as pl
from jax.experimental.pallas import tpu as pltpu
from jax.experimental.pallas import tpu_sc as plsc

N = 4096
K = 128
LANES = 16
NC, NS = 2, 16
NW = NC * NS
ROWS_PER_W = N // NW
G = 8
NGROUPS = ROWS_PER_W // G
STRIDE = N + 2 * LANES
STEPS = N // 2
UNROLL = 8
HBINS = 257

_SIGN = np.uint32(0x80000000)


def _sc_body(s_hbm, tot_hbm, diag_hbm, stag, buf, ebuf, hist_c, hist_f,
             tot_res, diag_res, sem):
    wid = lax.axis_index("s") * NC + lax.axis_index("c")
    row0 = wid * ROWS_PER_W
    lane_iota = lax.iota(jnp.int32, LANES)
    i8 = lane_iota & 7
    half = lane_iota >> 3
    ones_i = jnp.ones((LANES,), jnp.int32)
    zeros_i = jnp.zeros((LANES,), jnp.int32)
    zeros_f = jnp.zeros((LANES,), jnp.float32)
    hscat = lane_iota * HBINS
    hbase = i8 * HBINS
    hbase2 = (i8 + 8) * HBINS
    idx0 = i8 * (STRIDE + 2) + half

    def do_group(g, _):
        base_row = row0 + g * G

        pltpu.sync_copy(s_hbm.at[pl.ds(base_row, G)], stag)

        dval = plsc.load_gather(stag, [i8, i8 + base_row])
        ediag = jnp.exp(dval)

        for l in range(G):
            grow = base_row + l

            @plsc.parallel_loop(0, N // LANES, unroll=UNROLL)
            def _prep(j, l=l, grow=grow):
                c0 = j * LANES
                col = lane_iota + c0
                v = stag[l, pl.ds(c0, LANES)]
                isdiag = col == grow
                b = lax.bitcast_convert_type(v, jnp.uint32)
                u = jnp.where(b >= _SIGN, ~b, b | _SIGN)
                u = jnp.where(isdiag, jnp.uint32(0), u)
                buf[pl.ds(l * STRIDE + c0, LANES)] = lax.bitcast_convert_type(
                    u, jnp.float32)
                ebuf[pl.ds(l * STRIDE + c0, LANES)] = jnp.exp(v)

        for l in range(1, G):
            fu = buf[pl.ds(l * STRIDE, LANES)]
            fe = ebuf[pl.ds(l * STRIDE, LANES)]
            tm = lane_iota < 2 * l
            plsc.store_scatter(buf, [lane_iota + (l * STRIDE + N)], fu,
                               mask=tm)
            plsc.store_scatter(ebuf, [lane_iota + (l * STRIDE + N)], fe,
                               mask=tm)

        prefix = jnp.zeros((LANES,), jnp.uint32)
        k_rem = jnp.full((LANES,), K, jnp.int32)
        s_total = zeros_f

        for lvl in range(4):
            shift = np.uint32(24 - 8 * lvl)
            upper = np.uint32(0xFFFFFFFF) ^ np.uint32(
                (1 << (32 - 8 * lvl)) - 1)

            @plsc.parallel_loop(0, (LANES * HBINS) // LANES, unroll=8)
            def _zh(i):
                hist_c[pl.ds(i * LANES, LANES)] = zeros_i
                hist_f[pl.ds(i * LANES, LANES)] = zeros_f

            @plsc.parallel_loop(0, STEPS, unroll=UNROLL)
            def _hist(j, prefix=prefix, upper=upper, shift=shift, lvl=lvl):
                idx = idx0 + 2 * j
                u = lax.bitcast_convert_type(
                    plsc.load_gather(buf, [idx]), jnp.uint32)
                ev = plsc.load_gather(ebuf, [idx])
                byte = lax.shift_right_logical(u, shift) & jnp.uint32(0xFF)
                hidx = hscat + byte.astype(jnp.int32)
                if lvl == 0:
                    plsc.addupdate_scatter(hist_c, [hidx], ones_i)
                    plsc.addupdate_scatter(hist_f, [hidx], ev)
                else:
                    act = (u & upper) == prefix
                    plsc.addupdate_scatter(hist_c, [hidx], ones_i, mask=act)
                    plsc.addupdate_scatter(hist_f, [hidx], ev, mask=act)

            @plsc.parallel_loop(
                0, 256, unroll=4,
                carry=(zeros_i, zeros_f, zeros_i, zeros_i, zeros_f,
                       jnp.zeros((LANES,), jnp.bool_)))
            def _scan(i, carry):
                acc_c, acc_e, B, gt, s_lvl, done = carry
                b = 255 - i
                h = (plsc.load_gather(hist_c, [hbase + b])
                     + plsc.load_gather(hist_c, [hbase2 + b]))
                he = (plsc.load_gather(hist_f, [hbase + b])
                      + plsc.load_gather(hist_f, [hbase2 + b]))
                acc_c = acc_c + h
                crossed = acc_c >= k_rem
                newly = jnp.logical_and(crossed, jnp.logical_not(done))
                B = jnp.where(newly, b, B)
                gt = jnp.where(newly, acc_c - h, gt)
                s_lvl = jnp.where(newly, acc_e, s_lvl)
                acc_e = acc_e + he
                done = jnp.logical_or(done, crossed)
                return acc_c, acc_e, B, gt, s_lvl, done

            _, _, B, gt, s_lvl, _ = _scan

            prefix = prefix | lax.shift_left(
                B.astype(jnp.uint32), shift)
            k_rem = k_rem - gt
            s_total = s_total + s_lvl

        t_bits = jnp.where(prefix >= _SIGN, prefix & ~_SIGN, ~prefix)
        t_val = lax.bitcast_convert_type(t_bits, jnp.float32)
        neg = s_total + k_rem.astype(jnp.float32) * jnp.exp(t_val)
        total = neg + ediag

        out_idx = i8 + g * G
        m8 = lane_iota < 8
        plsc.store_scatter(tot_res, [out_idx], total, mask=m8)
        plsc.store_scatter(diag_res, [out_idx], dval, mask=m8)
        return 0

    lax.fori_loop(0, NGROUPS, do_group, 0, unroll=False)
    pltpu.sync_copy(tot_res, tot_hbm.at[pl.ds(row0, ROWS_PER_W)])
    pltpu.sync_copy(diag_res, diag_hbm.at[pl.ds(row0, ROWS_PER_W)])


@jax.jit
def _sc_select(s):
    mesh = plsc.VectorSubcoreMesh(core_axis_name="c", subcore_axis_name="s",
                                  num_cores=NC, num_subcores=NS)
    return pl.kernel(
        _sc_body,
        out_type=[
            jax.ShapeDtypeStruct((N,), jnp.float32),
            jax.ShapeDtypeStruct((N,), jnp.float32),
        ],
        mesh=mesh,
        compiler_params=pltpu.CompilerParams(needs_layout_passes=False),
        scratch_types=[
            pltpu.VMEM((G, N), jnp.float32),
            pltpu.VMEM((G * STRIDE,), jnp.float32),
            pltpu.VMEM((G * STRIDE,), jnp.float32),
            pltpu.VMEM((LANES * HBINS,), jnp.int32),
            pltpu.VMEM((LANES * HBINS,), jnp.float32),
            pltpu.VMEM((ROWS_PER_W,), jnp.float32),
            pltpu.VMEM((ROWS_PER_W,), jnp.float32),
            pltpu.SemaphoreType.DMA,
        ],
    )(s)


def _finish_body(tot_ref, diag_ref, out_ref):
    out_ref[0, 0] = jnp.mean(jnp.log(tot_ref[...]) - diag_ref[...])


@jax.jit
def _tc_finish(tot, diag):
    return pl.pallas_call(
        _finish_body,
        out_shape=jax.ShapeDtypeStruct((1, 1), jnp.float32),
        out_specs=pl.BlockSpec(memory_space=pltpu.SMEM),
    )(tot, diag)


def kernel(s):
    tot, diag = _sc_select(s)
    out = _tc_finish(tot.reshape(32, ROWS_PER_W), diag.reshape(32, ROWS_PER_W))
    return out[0, 0]


# speedup vs baseline: 4.5211x; 1.4417x over previous
import functools

import jax
import jax.numpy as jnp
import numpy as np
from jax import lax
from jax.experimental import pallas as pl
from jax.experimental.pallas import tpu as pltpu
from jax.experimental.pallas import tpu_sc as plsc

N = 4096
K = 128
LANES = 16
NC, NS = 2, 16
NW = NC * NS
ROWS_PER_W = N // NW
G = 8
NGROUPS = ROWS_PER_W // G
STRIDE = N + 2 * LANES
STEPS = N // 2
UNROLL = 8
HBINS = 257

_SIGN = np.uint32(0x80000000)


def _sc_body(s_hbm, tot_hbm, diag_hbm, stag, buf, hist_c, tmp16,
             tot_res, diag_res, sem):
    wid = lax.axis_index("s") * NC + lax.axis_index("c")
    row0 = wid * ROWS_PER_W
    lane_iota = lax.iota(jnp.int32, LANES)
    i8 = lane_iota & 7
    half = lane_iota >> 3
    ones_i = jnp.ones((LANES,), jnp.int32)
    zeros_i = jnp.zeros((LANES,), jnp.int32)
    zeros_f = jnp.zeros((LANES,), jnp.float32)
    hscat = lane_iota * HBINS
    hbase = i8 * HBINS
    hbase2 = (i8 + 8) * HBINS
    idx0 = i8 * (STRIDE + 2) + half

    def do_group(g, _):
        base_row = row0 + g * G

        pltpu.sync_copy(s_hbm.at[pl.ds(base_row, G)], stag)

        dval = plsc.load_gather(stag, [i8, i8 + base_row])
        ediag = jnp.exp(dval)

        for l in range(G):
            grow = base_row + l

            @plsc.parallel_loop(0, N // LANES, unroll=UNROLL)
            def _prep(j, l=l, grow=grow):
                c0 = j * LANES
                col = lane_iota + c0
                v = stag[l, pl.ds(c0, LANES)]
                isdiag = col == grow
                b = lax.bitcast_convert_type(v, jnp.uint32)
                u = jnp.where(b >= _SIGN, ~b, b | _SIGN)
                u = jnp.where(isdiag, jnp.uint32(0), u)
                buf[pl.ds(l * STRIDE + c0, LANES)] = lax.bitcast_convert_type(
                    u, jnp.float32)

        for l in range(1, G):
            fu = buf[pl.ds(l * STRIDE, LANES)]
            plsc.store_scatter(buf, [lane_iota + (l * STRIDE + N)], fu,
                               mask=lane_iota < 2 * l)

        prefix = jnp.zeros((LANES,), jnp.uint32)
        k_rem = jnp.full((LANES,), K, jnp.int32)

        for lvl in range(4):
            shift = np.uint32(24 - 8 * lvl)
            upper = np.uint32(0xFFFFFFFF) ^ np.uint32(
                (1 << (32 - 8 * lvl)) - 1)

            @plsc.parallel_loop(0, (LANES * HBINS) // LANES, unroll=8)
            def _zh(i):
                hist_c[pl.ds(i * LANES, LANES)] = zeros_i

            @plsc.parallel_loop(0, STEPS, unroll=UNROLL)
            def _hist(j, prefix=prefix, upper=upper, shift=shift, lvl=lvl):
                idx = idx0 + 2 * j
                u = lax.bitcast_convert_type(
                    plsc.load_gather(buf, [idx]), jnp.uint32)
                byte = lax.shift_right_logical(u, shift) & jnp.uint32(0xFF)
                hidx = hscat + byte.astype(jnp.int32)
                if lvl == 0:
                    plsc.addupdate_scatter(hist_c, [hidx], ones_i)
                else:
                    act = (u & upper) == prefix
                    plsc.addupdate_scatter(hist_c, [hidx], ones_i, mask=act)

            @plsc.parallel_loop(
                0, 256, unroll=4,
                carry=(zeros_i, zeros_i, zeros_i,
                       jnp.zeros((LANES,), jnp.bool_)))
            def _scan(i, carry):
                acc_c, B, gt, done = carry
                b = 255 - i
                h = (plsc.load_gather(hist_c, [hbase + b])
                     + plsc.load_gather(hist_c, [hbase2 + b]))
                acc_c = acc_c + h
                crossed = acc_c >= k_rem
                newly = jnp.logical_and(crossed, jnp.logical_not(done))
                B = jnp.where(newly, b, B)
                gt = jnp.where(newly, acc_c - h, gt)
                done = jnp.logical_or(done, crossed)
                return acc_c, B, gt, done

            _, B, gt, _ = _scan

            prefix = prefix | lax.shift_left(
                B.astype(jnp.uint32), shift)
            k_rem = k_rem - gt

        @plsc.parallel_loop(0, STEPS, unroll=UNROLL, carry=zeros_f)
        def _fsum(j, acc):
            idx = idx0 + 2 * j
            u = lax.bitcast_convert_type(
                plsc.load_gather(buf, [idx]), jnp.uint32)
            m = u > prefix
            bits = jnp.where(u >= _SIGN, u & ~_SIGN, ~u)
            e = jnp.exp(lax.bitcast_convert_type(bits, jnp.float32))
            return acc + jnp.where(m, e, 0.0)

        tmp16[pl.ds(0, LANES)] = _fsum
        s_total = (plsc.load_gather(tmp16, [i8])
                   + plsc.load_gather(tmp16, [i8 + 8]))

        t_bits = jnp.where(prefix >= _SIGN, prefix & ~_SIGN, ~prefix)
        t_val = lax.bitcast_convert_type(t_bits, jnp.float32)
        neg = s_total + k_rem.astype(jnp.float32) * jnp.exp(t_val)
        total = neg + ediag

        out_idx = i8 + g * G
        m8 = lane_iota < 8
        plsc.store_scatter(tot_res, [out_idx], total, mask=m8)
        plsc.store_scatter(diag_res, [out_idx], dval, mask=m8)
        return 0

    lax.fori_loop(0, NGROUPS, do_group, 0, unroll=False)
    pltpu.sync_copy(tot_res, tot_hbm.at[pl.ds(row0, ROWS_PER_W)])
    pltpu.sync_copy(diag_res, diag_hbm.at[pl.ds(row0, ROWS_PER_W)])


@jax.jit
def _sc_select(s):
    mesh = plsc.VectorSubcoreMesh(core_axis_name="c", subcore_axis_name="s",
                                  num_cores=NC, num_subcores=NS)
    return pl.kernel(
        _sc_body,
        out_type=[
            jax.ShapeDtypeStruct((N,), jnp.float32),
            jax.ShapeDtypeStruct((N,), jnp.float32),
        ],
        mesh=mesh,
        compiler_params=pltpu.CompilerParams(needs_layout_passes=False),
        scratch_types=[
            pltpu.VMEM((G, N), jnp.float32),
            pltpu.VMEM((G * STRIDE,), jnp.float32),
            pltpu.VMEM((LANES * HBINS,), jnp.int32),
            pltpu.VMEM((LANES,), jnp.float32),
            pltpu.VMEM((ROWS_PER_W,), jnp.float32),
            pltpu.VMEM((ROWS_PER_W,), jnp.float32),
            pltpu.SemaphoreType.DMA,
        ],
    )(s)


def _finish_body(tot_ref, diag_ref, out_ref):
    out_ref[0, 0] = jnp.mean(jnp.log(tot_ref[...]) - diag_ref[...])


@jax.jit
def _tc_finish(tot, diag):
    return pl.pallas_call(
        _finish_body,
        out_shape=jax.ShapeDtypeStruct((1, 1), jnp.float32),
        out_specs=pl.BlockSpec(memory_space=pltpu.SMEM),
    )(tot, diag)


def kernel(s):
    tot, diag = _sc_select(s)
    out = _tc_finish(tot.reshape(32, ROWS_PER_W), diag.reshape(32, ROWS_PER_W))
    return out[0, 0]


# speedup vs baseline: 4.8736x; 1.0780x over previous
import functools

import jax
import jax.numpy as jnp
import numpy as np
from jax import lax
from jax.experimental import pallas as pl
from jax.experimental.pallas import tpu as pltpu
from jax.experimental.pallas import tpu_sc as plsc

N = 4096
K = 128
LANES = 16
NC, NS = 2, 16
NW = NC * NS
ROWS_PER_W = N // NW
G = 8
NGROUPS = ROWS_PER_W // G
STRIDE = N + 2 * LANES
STEPS = N // 2
UNROLL = 8
HBINS = 257

_SIGN = np.uint32(0x80000000)


def _sc_body(s_hbm, tot_hbm, diag_hbm, stag, buf, hist_c, tmp16,
             tot_res, diag_res, sem):
    wid = lax.axis_index("s") * NC + lax.axis_index("c")
    row0 = wid * ROWS_PER_W
    lane_iota = lax.iota(jnp.int32, LANES)
    i8 = lane_iota & 7
    half = lane_iota >> 3
    ones_i = jnp.ones((LANES,), jnp.int32)
    zeros_i = jnp.zeros((LANES,), jnp.int32)
    zeros_f = jnp.zeros((LANES,), jnp.float32)
    hscat = lane_iota * HBINS
    hbase = i8 * HBINS
    hbase2 = (i8 + 8) * HBINS
    idx0 = i8 * (STRIDE + 2) + half

    def do_pair(jj, _):
      for p in range(2):
        g = jj * 2 + p
        base_row = row0 + g * G

        pltpu.make_async_copy(s_hbm.at[pl.ds(base_row, G)], stag.at[p],
                              sem).wait()
        nxt = jnp.minimum(base_row + G, N - G)
        pltpu.async_copy(s_hbm.at[pl.ds(nxt, G)], stag.at[1 - p], sem)

        dval = plsc.load_gather(stag, [jnp.zeros((LANES,), jnp.int32) + p,
                                       i8, i8 + base_row])
        ediag = jnp.exp(dval)

        for l in range(G):
            grow = base_row + l

            @plsc.parallel_loop(0, N // LANES, unroll=UNROLL)
            def _prep(j, l=l, grow=grow, p=p):
                c0 = j * LANES
                col = lane_iota + c0
                v = stag[p, l, pl.ds(c0, LANES)]
                isdiag = col == grow
                b = lax.bitcast_convert_type(v, jnp.uint32)
                u = jnp.where(b >= _SIGN, ~b, b | _SIGN)
                u = jnp.where(isdiag, jnp.uint32(0), u)
                buf[pl.ds(l * STRIDE + c0, LANES)] = lax.bitcast_convert_type(
                    u, jnp.float32)

        for l in range(1, G):
            fu = buf[pl.ds(l * STRIDE, LANES)]
            plsc.store_scatter(buf, [lane_iota + (l * STRIDE + N)], fu,
                               mask=lane_iota < 2 * l)

        prefix = jnp.zeros((LANES,), jnp.uint32)
        k_rem = jnp.full((LANES,), K, jnp.int32)

        for lvl in range(4):
            shift = np.uint32(24 - 8 * lvl)
            upper = np.uint32(0xFFFFFFFF) ^ np.uint32(
                (1 << (32 - 8 * lvl)) - 1)

            @plsc.parallel_loop(0, (LANES * HBINS) // LANES, unroll=8)
            def _zh(i):
                hist_c[pl.ds(i * LANES, LANES)] = zeros_i

            @plsc.parallel_loop(0, STEPS, unroll=UNROLL)
            def _hist(j, prefix=prefix, upper=upper, shift=shift, lvl=lvl):
                idx = idx0 + 2 * j
                u = lax.bitcast_convert_type(
                    plsc.load_gather(buf, [idx]), jnp.uint32)
                byte = lax.shift_right_logical(u, shift) & jnp.uint32(0xFF)
                hidx = hscat + byte.astype(jnp.int32)
                if lvl == 0:
                    plsc.addupdate_scatter(hist_c, [hidx], ones_i)
                else:
                    act = (u & upper) == prefix
                    plsc.addupdate_scatter(hist_c, [hidx], ones_i, mask=act)

            @plsc.parallel_loop(
                0, 256, unroll=4,
                carry=(zeros_i, zeros_i, zeros_i,
                       jnp.zeros((LANES,), jnp.bool_)))
            def _scan(i, carry):
                acc_c, B, gt, done = carry
                b = 255 - i
                h = (plsc.load_gather(hist_c, [hbase + b])
                     + plsc.load_gather(hist_c, [hbase2 + b]))
                acc_c = acc_c + h
                crossed = acc_c >= k_rem
                newly = jnp.logical_and(crossed, jnp.logical_not(done))
                B = jnp.where(newly, b, B)
                gt = jnp.where(newly, acc_c - h, gt)
                done = jnp.logical_or(done, crossed)
                return acc_c, B, gt, done

            _, B, gt, _ = _scan

            prefix = prefix | lax.shift_left(
                B.astype(jnp.uint32), shift)
            k_rem = k_rem - gt

        @plsc.parallel_loop(0, STEPS, unroll=UNROLL, carry=zeros_f)
        def _fsum(j, acc):
            idx = idx0 + 2 * j
            u = lax.bitcast_convert_type(
                plsc.load_gather(buf, [idx]), jnp.uint32)
            m = u > prefix
            bits = jnp.where(u >= _SIGN, u & ~_SIGN, ~u)
            e = jnp.exp(lax.bitcast_convert_type(bits, jnp.float32))
            return acc + jnp.where(m, e, 0.0)

        tmp16[pl.ds(0, LANES)] = _fsum
        s_total = (plsc.load_gather(tmp16, [i8])
                   + plsc.load_gather(tmp16, [i8 + 8]))

        t_bits = jnp.where(prefix >= _SIGN, prefix & ~_SIGN, ~prefix)
        t_val = lax.bitcast_convert_type(t_bits, jnp.float32)
        neg = s_total + k_rem.astype(jnp.float32) * jnp.exp(t_val)
        total = neg + ediag

        out_idx = i8 + g * G
        m8 = lane_iota < 8
        plsc.store_scatter(tot_res, [out_idx], total, mask=m8)
        plsc.store_scatter(diag_res, [out_idx], dval, mask=m8)
      return 0

    pltpu.async_copy(s_hbm.at[pl.ds(row0, G)], stag.at[0], sem)
    lax.fori_loop(0, NGROUPS // 2, do_pair, 0, unroll=False)
    pltpu.make_async_copy(s_hbm.at[pl.ds(N - G, G)], stag.at[1],
                          sem).wait()
    pltpu.sync_copy(tot_res, tot_hbm.at[pl.ds(row0, ROWS_PER_W)])
    pltpu.sync_copy(diag_res, diag_hbm.at[pl.ds(row0, ROWS_PER_W)])


@jax.jit
def _sc_select(s):
    mesh = plsc.VectorSubcoreMesh(core_axis_name="c", subcore_axis_name="s",
                                  num_cores=NC, num_subcores=NS)
    return pl.kernel(
        _sc_body,
        out_type=[
            jax.ShapeDtypeStruct((N,), jnp.float32),
            jax.ShapeDtypeStruct((N,), jnp.float32),
        ],
        mesh=mesh,
        compiler_params=pltpu.CompilerParams(needs_layout_passes=False),
        scratch_types=[
            pltpu.VMEM((2, G, N), jnp.float32),
            pltpu.VMEM((G * STRIDE,), jnp.float32),
            pltpu.VMEM((LANES * HBINS,), jnp.int32),
            pltpu.VMEM((LANES,), jnp.float32),
            pltpu.VMEM((ROWS_PER_W,), jnp.float32),
            pltpu.VMEM((ROWS_PER_W,), jnp.float32),
            pltpu.SemaphoreType.DMA,
        ],
    )(s)


def _finish_body(tot_ref, diag_ref, out_ref):
    out_ref[0, 0] = jnp.mean(jnp.log(tot_ref[...]) - diag_ref[...])


@jax.jit
def _tc_finish(tot, diag):
    return pl.pallas_call(
        _finish_body,
        out_shape=jax.ShapeDtypeStruct((1, 1), jnp.float32),
        out_specs=pl.BlockSpec(memory_space=pltpu.SMEM),
    )(tot, diag)


def kernel(s):
    tot, diag = _sc_select(s)
    out = _tc_finish(tot.reshape(32, ROWS_PER_W), diag.reshape(32, ROWS_PER_W))
    return out[0, 0]
